# baseline (speedup 1.0000x reference)
import functools

import jax
import jax.numpy as jnp
from jax.experimental import pallas as pl
from jax.experimental.pallas import tpu as pltpu
from jax.experimental.pallas import tpu_sc as plsc

_B = 4096
_N = 1000
_BBLK = 2048
_GRID = _B // _BBLK
_NC = 2
_NS = 16
_NW = _NC * _NS
_SPW = _B // _NW
_L = 16


def _tc_argmax_body(x0_ref, x1_ref, a_ref, b_ref):
    a_ref[...] = jnp.argmax(x0_ref[0], axis=0).astype(jnp.int32).reshape(1, -1)
    b_ref[...] = jnp.argmax(x1_ref[0], axis=0).astype(jnp.int32).reshape(1, -1)


def _tc_argmax(xt):
    return pl.pallas_call(
        _tc_argmax_body,
        grid=(_GRID,),
        in_specs=[
            pl.BlockSpec((1, _N, _BBLK), lambda i: (0, 0, i)),
            pl.BlockSpec((1, _N, _BBLK), lambda i: (1, 0, i)),
        ],
        out_specs=[
            pl.BlockSpec((1, _BBLK), lambda i: (0, i)),
            pl.BlockSpec((1, _BBLK), lambda i: (0, i)),
        ],
        out_shape=[
            jax.ShapeDtypeStruct((1, _B), jnp.int32),
            jax.ShapeDtypeStruct((1, _B), jnp.int32),
        ],
    )(xt, xt)


def _sc_pair_body(a_hbm, b_hbm, tabf_hbm, tbaf_hbm, out_hbm,
                  ia_v, ib_v, pa_v, pb_v, g3_v, g6_v, acc_v, sem):
    wid = jax.lax.axis_index("s") * _NC + jax.lax.axis_index("c")
    base = wid * _SPW
    ca = pltpu.make_async_copy(a_hbm.at[pl.ds(base, _SPW)], ia_v, sem)
    cb = pltpu.make_async_copy(b_hbm.at[pl.ds(base, _SPW)], ib_v, sem)
    ca.start()
    cb.start()
    ca.wait()
    cb.wait()

    for k in range(_SPW // _L):
        av = ia_v[pl.ds(k * _L, _L)]
        bv = ib_v[pl.ds(k * _L, _L)]
        pa_v[pl.ds(k * _L, _L)] = bv * _N + av
        pb_v[pl.ds(k * _L, _L)] = av * _N + bv

    c3 = pltpu.make_async_copy(tabf_hbm.at[pa_v], g3_v, sem)
    c6 = pltpu.make_async_copy(tbaf_hbm.at[pb_v], g6_v, sem)
    c3.start()
    c6.start()
    c3.wait()
    c6.wait()

    def accum(g_ref):
        acc = jnp.zeros((_L,), jnp.float32)
        for k in range(_SPW // _L):
            acc = acc + g_ref[pl.ds(k * _L, _L)]
        return acc

    acc_v[pl.ds(0, _L)] = accum(g3_v)
    acc_v[pl.ds(_L, _L)] = accum(g6_v)
    pltpu.sync_copy(acc_v, out_hbm.at[wid])


def _sc_pair(a_i, b_i, tabf, tbaf):
    mesh = plsc.VectorSubcoreMesh(core_axis_name="c", subcore_axis_name="s",
                                  num_cores=_NC, num_subcores=_NS)
    f = functools.partial(
        pl.kernel,
        out_type=jax.ShapeDtypeStruct((_NW, 2 * _L), jnp.float32),
        mesh=mesh,
        scratch_types=[
            pltpu.VMEM((_SPW,), jnp.int32),
            pltpu.VMEM((_SPW,), jnp.int32),
            pltpu.VMEM((_SPW,), jnp.int32),
            pltpu.VMEM((_SPW,), jnp.int32),
            pltpu.VMEM((_SPW,), jnp.float32),
            pltpu.VMEM((_SPW,), jnp.float32),
            pltpu.VMEM((2 * _L,), jnp.float32),
            pltpu.SemaphoreType.DMA,
        ],
    )(_sc_pair_body)
    return f(a_i, b_i, tabf, tbaf)


_SGRID = 8
_TROWS = 128
_HCH = 128


def _tc_stats_body(tab_ref, tba_ref, p1ab_ref, p1ba_ref, a_ref, b_ref,
                   misc_ref, mab_v, sab_v, mba_v, sba_v, ha_v, hb_v):
    i = pl.program_id(0)

    @pl.when(i == 0)
    def _init():
        for m_v, s_v in ((mab_v, sab_v), (mba_v, sba_v)):
            m_v[...] = jnp.full((1, _N), -3.0e38, jnp.float32)
            s_v[...] = jnp.zeros((1, _N), jnp.float32)
        for h_v in (ha_v, hb_v):
            h_v[...] = jnp.zeros((1, _N), jnp.float32)

    liota = jax.lax.broadcasted_iota(jnp.int32, (1, _N), 1)
    for idx_ref, h_v in ((a_ref, ha_v), (b_ref, hb_v)):
        col = idx_ref[0].reshape(_B // _SGRID, 1)
        h = h_v[...]
        for c in range(_B // _SGRID // _HCH):
            blk = jax.lax.slice(col, (c * _HCH, 0), ((c + 1) * _HCH, 1))
            eq = (blk == liota).astype(jnp.float32)
            h = h + jnp.sum(eq, axis=0, keepdims=True)
        h_v[...] = h

    rowbase = i * _TROWS
    riota = jax.lax.broadcasted_iota(jnp.int32, (_TROWS, _N), 0)
    valid = riota < (_N - rowbase)
    for t_ref, m_v, s_v in ((tab_ref, mab_v, sab_v), (tba_ref, mba_v, sba_v)):
        t = jnp.where(valid, t_ref[...], -3.0e38)
        bm = jnp.max(t, axis=0, keepdims=True)
        mnew = jnp.maximum(m_v[...], bm)
        bsum = jnp.sum(jnp.exp(t - mnew), axis=0, keepdims=True)
        s_v[...] = s_v[...] * jnp.exp(m_v[...] - mnew) + bsum
        m_v[...] = mnew

    @pl.when(i == _SGRID - 1)
    def _final():
        def lse1(p_ref):
            p = p_ref[...]
            m = jnp.max(p)
            return m + jnp.log(jnp.sum(jnp.exp(p - m)))

        terms = []
        for p1_ref, m_v, s_v, h_v in ((p1ab_ref, mab_v, sab_v, ha_v),
                                      (p1ba_ref, mba_v, sba_v, hb_v)):
            cn = m_v[...] + jnp.log(s_v[...])
            terms.append(jnp.sum((p1_ref[...] - cn) * h_v[...]))
            terms.append(lse1(p1_ref))

        lane = jax.lax.broadcasted_iota(jnp.int32, (1, 8), 1)
        misc_ref[...] = (jnp.where(lane == 0, terms[0], 0.0)
                         + jnp.where(lane == 1, terms[1], 0.0)
                         + jnp.where(lane == 2, terms[2], 0.0)
                         + jnp.where(lane == 3, terms[3], 0.0))


def _tc_stats(tab, tba, p1ab2, p1ba2, a2, b2):
    return pl.pallas_call(
        _tc_stats_body,
        grid=(_SGRID,),
        in_specs=[
            pl.BlockSpec((_TROWS, _N), lambda i: (i, 0)),
            pl.BlockSpec((_TROWS, _N), lambda i: (i, 0)),
            pl.BlockSpec((1, _N), lambda i: (0, 0)),
            pl.BlockSpec((1, _N), lambda i: (0, 0)),
            pl.BlockSpec((1, _B // _SGRID), lambda i: (0, i)),
            pl.BlockSpec((1, _B // _SGRID), lambda i: (0, i)),
        ],
        out_specs=pl.BlockSpec((1, 8), lambda i: (0, 0)),
        out_shape=jax.ShapeDtypeStruct((1, 8), jnp.float32),
        scratch_shapes=[
            pltpu.VMEM((1, _N), jnp.float32),
            pltpu.VMEM((1, _N), jnp.float32),
            pltpu.VMEM((1, _N), jnp.float32),
            pltpu.VMEM((1, _N), jnp.float32),
            pltpu.VMEM((1, _N), jnp.float32),
            pltpu.VMEM((1, _N), jnp.float32),
        ],
    )(tab, tba, p1ab2, p1ba2, a2, b2)


def _tc_combine_body(part_ref, misc_ref, gamma_ref, out_ref):
    col = jax.lax.broadcasted_iota(jnp.int32, (_NW, 2 * _L), 1)
    part = part_ref[...]
    pa = jnp.sum(jnp.where(col < _L, part, 0.0))
    pb = jnp.sum(jnp.where(col >= _L, part, 0.0))

    def bc(x):
        return jnp.full((1, 8), x, jnp.float32)

    g0 = bc(gamma_ref[0, 0])
    g1 = bc(gamma_ref[0, 1])
    mg = jnp.maximum(g0, g1)
    lseg = mg + jnp.log(jnp.exp(g0 - mg) + jnp.exp(g1 - mg))
    m_ab = ((g0 - lseg) + bc(pa) + bc(misc_ref[0, 0])
            - jnp.float32(_B) * bc(misc_ref[0, 1]))
    m_ba = ((g1 - lseg) + bc(pb) + bc(misc_ref[0, 2])
            - jnp.float32(_B) * bc(misc_ref[0, 3]))
    mm = jnp.maximum(m_ab, m_ba)
    out_ref[...] = mm + jnp.log(jnp.exp(m_ab - mm) + jnp.exp(m_ba - mm))


def _tc_combine(part, misc, gamma2):
    return pl.pallas_call(
        _tc_combine_body,
        out_shape=jax.ShapeDtypeStruct((1, 8), jnp.float32),
    )(part, misc, gamma2)


def kernel(samples, P_1_AB, P_2_1_AB, P_1_BA, P_2_1_BA, gamma):
    b, mdim, n = samples.shape
    xt = jnp.transpose(samples, (1, 2, 0))
    a2, b2 = _tc_argmax(xt)
    part = _sc_pair(a2.reshape(b), b2.reshape(b),
                    P_2_1_AB.reshape(n * n), P_2_1_BA.reshape(n * n))
    misc = _tc_stats(P_2_1_AB, P_2_1_BA,
                     P_1_AB.reshape(1, n), P_1_BA.reshape(1, n), a2, b2)
    out = _tc_combine(part, misc, gamma.reshape(1, 2))
    return out[0, 0]


# speedup vs baseline: 1.0188x; 1.0188x over previous
import functools

import jax
import jax.numpy as jnp
from jax.experimental import pallas as pl
from jax.experimental.pallas import tpu as pltpu
from jax.experimental.pallas import tpu_sc as plsc

_B = 4096
_N = 1000
_BBLK = 1024
_GRID = _B // _BBLK
_NC = 2
_NS = 16
_NW = _NC * _NS
_SPW = _B // _NW
_L = 16


def _tc_argmax_body(x0_ref, x1_ref, a_ref, b_ref):
    a_ref[...] = jnp.argmax(x0_ref[0], axis=0).astype(jnp.int32).reshape(1, -1)
    b_ref[...] = jnp.argmax(x1_ref[0], axis=0).astype(jnp.int32).reshape(1, -1)


def _tc_argmax(xt):
    return pl.pallas_call(
        _tc_argmax_body,
        grid=(_GRID,),
        in_specs=[
            pl.BlockSpec((1, _N, _BBLK), lambda i: (0, 0, i)),
            pl.BlockSpec((1, _N, _BBLK), lambda i: (1, 0, i)),
        ],
        out_specs=[
            pl.BlockSpec((1, _BBLK), lambda i: (0, i)),
            pl.BlockSpec((1, _BBLK), lambda i: (0, i)),
        ],
        out_shape=[
            jax.ShapeDtypeStruct((1, _B), jnp.int32),
            jax.ShapeDtypeStruct((1, _B), jnp.int32),
        ],
    )(xt, xt)


def _sc_pair_body(a_hbm, b_hbm, tabf_hbm, tbaf_hbm, out_hbm,
                  ia_v, ib_v, pa_v, pb_v, g3_v, g6_v, acc_v, sem):
    wid = jax.lax.axis_index("s") * _NC + jax.lax.axis_index("c")
    base = wid * _SPW
    ca = pltpu.make_async_copy(a_hbm.at[pl.ds(base, _SPW)], ia_v, sem)
    cb = pltpu.make_async_copy(b_hbm.at[pl.ds(base, _SPW)], ib_v, sem)
    ca.start()
    cb.start()
    ca.wait()
    cb.wait()

    for k in range(_SPW // _L):
        av = ia_v[pl.ds(k * _L, _L)]
        bv = ib_v[pl.ds(k * _L, _L)]
        pa_v[pl.ds(k * _L, _L)] = bv * _N + av
        pb_v[pl.ds(k * _L, _L)] = av * _N + bv

    c3 = pltpu.make_async_copy(tabf_hbm.at[pa_v], g3_v, sem)
    c6 = pltpu.make_async_copy(tbaf_hbm.at[pb_v], g6_v, sem)
    c3.start()
    c6.start()
    c3.wait()
    c6.wait()

    def accum(g_ref):
        acc = jnp.zeros((_L,), jnp.float32)
        for k in range(_SPW // _L):
            acc = acc + g_ref[pl.ds(k * _L, _L)]
        return acc

    acc_v[pl.ds(0, _L)] = accum(g3_v)
    acc_v[pl.ds(_L, _L)] = accum(g6_v)
    pltpu.sync_copy(acc_v, out_hbm.at[wid])


def _sc_pair(a_i, b_i, tabf, tbaf):
    mesh = plsc.VectorSubcoreMesh(core_axis_name="c", subcore_axis_name="s",
                                  num_cores=_NC, num_subcores=_NS)
    f = functools.partial(
        pl.kernel,
        out_type=jax.ShapeDtypeStruct((_NW, 2 * _L), jnp.float32),
        mesh=mesh,
        scratch_types=[
            pltpu.VMEM((_SPW,), jnp.int32),
            pltpu.VMEM((_SPW,), jnp.int32),
            pltpu.VMEM((_SPW,), jnp.int32),
            pltpu.VMEM((_SPW,), jnp.int32),
            pltpu.VMEM((_SPW,), jnp.float32),
            pltpu.VMEM((_SPW,), jnp.float32),
            pltpu.VMEM((2 * _L,), jnp.float32),
            pltpu.SemaphoreType.DMA,
        ],
    )(_sc_pair_body)
    return f(a_i, b_i, tabf, tbaf)


_SGRID = 8
_TROWS = 128
_HCH = 128


def _tc_stats_body(tab_ref, tba_ref, p1ab_ref, p1ba_ref, a_ref, b_ref,
                   misc_ref, mab_v, sab_v, mba_v, sba_v, ha_v, hb_v):
    i = pl.program_id(0)

    @pl.when(i == 0)
    def _init():
        for m_v, s_v in ((mab_v, sab_v), (mba_v, sba_v)):
            m_v[...] = jnp.full((1, _N), -3.0e38, jnp.float32)
            s_v[...] = jnp.zeros((1, _N), jnp.float32)
        for h_v in (ha_v, hb_v):
            h_v[...] = jnp.zeros((1, _N), jnp.float32)

    liota = jax.lax.broadcasted_iota(jnp.int32, (1, _N), 1)
    for idx_ref, h_v in ((a_ref, ha_v), (b_ref, hb_v)):
        col = idx_ref[0].reshape(_B // _SGRID, 1)
        h = h_v[...]
        for c in range(_B // _SGRID // _HCH):
            blk = jax.lax.slice(col, (c * _HCH, 0), ((c + 1) * _HCH, 1))
            eq = (blk == liota).astype(jnp.float32)
            h = h + jnp.sum(eq, axis=0, keepdims=True)
        h_v[...] = h

    rowbase = i * _TROWS
    riota = jax.lax.broadcasted_iota(jnp.int32, (_TROWS, _N), 0)
    valid = riota < (_N - rowbase)
    for t_ref, m_v, s_v in ((tab_ref, mab_v, sab_v), (tba_ref, mba_v, sba_v)):
        t = jnp.where(valid, t_ref[...], -3.0e38)
        bm = jnp.max(t, axis=0, keepdims=True)
        mnew = jnp.maximum(m_v[...], bm)
        bsum = jnp.sum(jnp.exp(t - mnew), axis=0, keepdims=True)
        s_v[...] = s_v[...] * jnp.exp(m_v[...] - mnew) + bsum
        m_v[...] = mnew

    @pl.when(i == _SGRID - 1)
    def _final():
        def lse1(p_ref):
            p = p_ref[...]
            m = jnp.max(p)
            return m + jnp.log(jnp.sum(jnp.exp(p - m)))

        terms = []
        for p1_ref, m_v, s_v, h_v in ((p1ab_ref, mab_v, sab_v, ha_v),
                                      (p1ba_ref, mba_v, sba_v, hb_v)):
            cn = m_v[...] + jnp.log(s_v[...])
            terms.append(jnp.sum((p1_ref[...] - cn) * h_v[...]))
            terms.append(lse1(p1_ref))

        lane = jax.lax.broadcasted_iota(jnp.int32, (1, 8), 1)
        misc_ref[...] = (jnp.where(lane == 0, terms[0], 0.0)
                         + jnp.where(lane == 1, terms[1], 0.0)
                         + jnp.where(lane == 2, terms[2], 0.0)
                         + jnp.where(lane == 3, terms[3], 0.0))


def _tc_stats(tab, tba, p1ab2, p1ba2, a2, b2):
    return pl.pallas_call(
        _tc_stats_body,
        grid=(_SGRID,),
        in_specs=[
            pl.BlockSpec((_TROWS, _N), lambda i: (i, 0)),
            pl.BlockSpec((_TROWS, _N), lambda i: (i, 0)),
            pl.BlockSpec((1, _N), lambda i: (0, 0)),
            pl.BlockSpec((1, _N), lambda i: (0, 0)),
            pl.BlockSpec((1, _B // _SGRID), lambda i: (0, i)),
            pl.BlockSpec((1, _B // _SGRID), lambda i: (0, i)),
        ],
        out_specs=pl.BlockSpec((1, 8), lambda i: (0, 0)),
        out_shape=jax.ShapeDtypeStruct((1, 8), jnp.float32),
        scratch_shapes=[
            pltpu.VMEM((1, _N), jnp.float32),
            pltpu.VMEM((1, _N), jnp.float32),
            pltpu.VMEM((1, _N), jnp.float32),
            pltpu.VMEM((1, _N), jnp.float32),
            pltpu.VMEM((1, _N), jnp.float32),
            pltpu.VMEM((1, _N), jnp.float32),
        ],
    )(tab, tba, p1ab2, p1ba2, a2, b2)


def _tc_combine_body(part_ref, misc_ref, gamma_ref, out_ref):
    col = jax.lax.broadcasted_iota(jnp.int32, (_NW, 2 * _L), 1)
    part = part_ref[...]
    pa = jnp.sum(jnp.where(col < _L, part, 0.0))
    pb = jnp.sum(jnp.where(col >= _L, part, 0.0))

    def bc(x):
        return jnp.full((1, 8), x, jnp.float32)

    g0 = bc(gamma_ref[0, 0])
    g1 = bc(gamma_ref[0, 1])
    mg = jnp.maximum(g0, g1)
    lseg = mg + jnp.log(jnp.exp(g0 - mg) + jnp.exp(g1 - mg))
    m_ab = ((g0 - lseg) + bc(pa) + bc(misc_ref[0, 0])
            - jnp.float32(_B) * bc(misc_ref[0, 1]))
    m_ba = ((g1 - lseg) + bc(pb) + bc(misc_ref[0, 2])
            - jnp.float32(_B) * bc(misc_ref[0, 3]))
    mm = jnp.maximum(m_ab, m_ba)
    out_ref[...] = mm + jnp.log(jnp.exp(m_ab - mm) + jnp.exp(m_ba - mm))


def _tc_combine(part, misc, gamma2):
    return pl.pallas_call(
        _tc_combine_body,
        out_shape=jax.ShapeDtypeStruct((1, 8), jnp.float32),
    )(part, misc, gamma2)


def kernel(samples, P_1_AB, P_2_1_AB, P_1_BA, P_2_1_BA, gamma):
    b, mdim, n = samples.shape
    xt = jnp.transpose(samples, (1, 2, 0))
    a2, b2 = _tc_argmax(xt)
    part = _sc_pair(a2.reshape(b), b2.reshape(b),
                    P_2_1_AB.reshape(n * n), P_2_1_BA.reshape(n * n))
    misc = _tc_stats(P_2_1_AB, P_2_1_BA,
                     P_1_AB.reshape(1, n), P_1_BA.reshape(1, n), a2, b2)
    out = _tc_combine(part, misc, gamma.reshape(1, 2))
    return out[0, 0]
